# baseline (speedup 1.0000x reference)
import jax
import jax.numpy as jnp
from jax import lax
from jax.experimental import pallas as pl
from jax.experimental.pallas import tpu as pltpu, tpu_sc as plsc

B = 4096
SEQ_LEN = 256
N_BINS = 256
LATENT_DIM = 64
DPAD = 128

NUM_CORES = 2
NUM_SUBCORES = 16
NW = NUM_CORES * NUM_SUBCORES
BPW = B // NW
STEP = 128
NSTEPS = 2 * BPW
BLKB = 8
BLKS = 2 * BLKB
NBLK = BPW // BLKB
CG = LATENT_DIM // 16


def _body(xs, token_tab, scale_tab, pos_tab, out,
          pos_v, gbuf, sbuf, obuf, idx_v, tok_sh, scl_sh,
          sem_gt0, sem_gt1, sem_gs0, sem_gs1, sem_o0, sem_o1, sem_i):
    sid = lax.axis_index("s")
    wid = sid * NUM_CORES + lax.axis_index("c")
    batch_base = wid * BPW
    sem_gt = (sem_gt0, sem_gt1)
    sem_gs = (sem_gs0, sem_gs1)
    sem_o = (sem_o0, sem_o1)

    @pl.when(sid == 0)
    def _stage_tables():
        pltpu.sync_copy(token_tab, tok_sh)
        pltpu.sync_copy(scale_tab, scl_sh)

    pltpu.sync_copy(pos_tab, pos_v)
    pltpu.sync_copy(xs.at[pl.ds(batch_base * 4, 4 * BLKB)], idx_v.at[0])
    plsc.subcore_barrier()

    def issue_tok(h, ls, p):
        pltpu.async_copy(tok_sh.at[idx_v.at[h, 2 * ls - (ls % 2)]],
                         gbuf.at[p], sem_gt[p])

    def issue_scl(h, ls, p):
        pltpu.async_copy(scl_sh.at[idx_v.at[h, 2 * ls - (ls % 2) + 2]],
                         sbuf.at[p], sem_gs[p])

    def blk_body(blk, _):
        h = lax.rem(blk, 2)

        @pl.when(blk > 0)
        def _wait_idx():
            pltpu.make_async_copy(xs.at[pl.ds(0, 4 * BLKB)], idx_v.at[h],
                                  sem_i).wait()

        @pl.when(blk + 1 < NBLK)
        def _prefetch_idx():
            nxt = (batch_base + (blk + 1) * BLKB) * 4
            pltpu.async_copy(xs.at[pl.ds(nxt, 4 * BLKB)], idx_v.at[1 - h], sem_i)

        for ls0 in (0, 1):
            issue_tok(h, ls0, ls0)
            issue_scl(h, ls0, ls0)

        def q_body(q, _):
            for p in range(2):
                ls = q * 2 + p
                g = blk * BLKS + ls
                pltpu.make_async_copy(token_tab.at[pl.ds(0, STEP)],
                                      gbuf.at[p], sem_gt[p]).wait()
                pltpu.make_async_copy(scale_tab.at[pl.ds(0, STEP)],
                                      sbuf.at[p], sem_gs[p]).wait()

                @pl.when(g >= 2)
                def _drain_out():
                    pltpu.make_async_copy(out.at[0, pl.ds(0, STEP)],
                                          obuf.at[p], sem_o[p]).wait()

                def row_body(rp, _):
                    for half in range(2):
                        r = 2 * rp + half
                        for c in range(CG):
                            sl = pl.ds(c * 16, 16)
                            v = (gbuf[p, r, sl] + sbuf[p, r, sl]
                                 + pos_v[p * (SEQ_LEN // 4) + rp,
                                         pl.ds(half * 64 + c * 16, 16)])
                            obuf[p, r, sl] = v
                    return 0

                lax.fori_loop(0, STEP // 2, row_body, 0)

                pltpu.async_copy(
                    obuf.at[p],
                    out.at[batch_base + lax.div(g, 2),
                           pl.ds(lax.rem(g, 2) * STEP, STEP)],
                    sem_o[p])

                @pl.when(ls + 2 < BLKS)
                def _prefetch_gathers():
                    issue_tok(h, ls + 2, p)
                    issue_scl(h, ls + 2, p)
            return 0

        lax.fori_loop(0, BLKS // 2, q_body, 0)
        return 0

    lax.fori_loop(0, NBLK, blk_body, 0)

    pltpu.make_async_copy(out.at[0, pl.ds(0, STEP)], obuf.at[0], sem_o0).wait()
    pltpu.make_async_copy(out.at[0, pl.ds(0, STEP)], obuf.at[1], sem_o1).wait()


@jax.jit
def _run(xs, token_pad, scale_pad, pos_pack):
    mesh = plsc.VectorSubcoreMesh(core_axis_name="c", subcore_axis_name="s")
    kfn = pl.kernel(
        _body,
        out_type=jax.ShapeDtypeStruct((B, SEQ_LEN, LATENT_DIM), jnp.float32),
        mesh=mesh,
        compiler_params=pltpu.CompilerParams(use_tc_tiling_on_sc=True),
        scratch_types=[
            pltpu.VMEM((SEQ_LEN // 2, DPAD), jnp.float32),
            pltpu.VMEM((2, STEP, DPAD), jnp.float32),
            pltpu.VMEM((2, STEP, DPAD), jnp.float32),
            pltpu.VMEM((2, STEP, LATENT_DIM), jnp.float32),
            pltpu.VMEM((2, 4 * BLKB, STEP), jnp.int32),
            pltpu.VMEM_SHARED((N_BINS, DPAD), jnp.float32),
            pltpu.VMEM_SHARED((N_BINS, DPAD), jnp.float32),
            pltpu.SemaphoreType.DMA,
            pltpu.SemaphoreType.DMA,
            pltpu.SemaphoreType.DMA,
            pltpu.SemaphoreType.DMA,
            pltpu.SemaphoreType.DMA,
            pltpu.SemaphoreType.DMA,
            pltpu.SemaphoreType.DMA,
        ],
    )
    return kfn(xs, token_pad, scale_pad, pos_pack)


def kernel(x, token_table, scale_table, pos_table):
    xs = x.reshape(B * 4, STEP)
    token_pad = jnp.pad(token_table, ((0, 0), (0, DPAD - LATENT_DIM)))
    scale_pad = jnp.pad(scale_table, ((0, 0), (0, DPAD - LATENT_DIM)))
    pos_pack = pos_table.reshape(SEQ_LEN // 2, DPAD)
    return _run(xs, token_pad, scale_pad, pos_pack)
